# baseline (speedup 1.0000x reference)
import jax
import jax.numpy as jnp
from jax.experimental import pallas as pl
from jax.experimental.pallas import tpu as pltpu

_CK = 128


def _pool_conv_broadcast_kernel(nk, x_ref, w_ref, b_ref, o_ref, acc_ref):
    k = pl.program_id(1)

    @pl.when(k == 0)
    def _init():
        acc_ref[...] = jnp.zeros_like(acc_ref)

    m = jnp.sum(x_ref[0], axis=1, keepdims=True)
    acc_ref[...] += jnp.dot(w_ref[...], m,
                            preferred_element_type=jnp.float32)

    @pl.when(k == nk - 1)
    def _emit():
        hw = o_ref.shape[2]
        pooled = acc_ref[...] * (1.0 / hw) + b_ref[...]
        o_ref[0] = jnp.broadcast_to(pooled, o_ref.shape[1:])


def kernel(features, depth, W, b):
    del depth
    N, C, H, Wd = features.shape
    O = W.shape[0]
    HW = H * Wd
    nk = C // _CK
    x = features.reshape(N, C, HW)
    w2 = W.reshape(O, C)
    b2 = b.reshape(O, 1)
    import functools
    out = pl.pallas_call(
        functools.partial(_pool_conv_broadcast_kernel, nk),
        grid=(N, nk),
        in_specs=[
            pl.BlockSpec((1, _CK, HW), lambda i, k: (i, k, 0)),
            pl.BlockSpec((O, _CK), lambda i, k: (0, k)),
            pl.BlockSpec((O, 1), lambda i, k: (0, 0)),
        ],
        out_specs=pl.BlockSpec((1, O, HW), lambda i, k: (i, 0, 0)),
        out_shape=jax.ShapeDtypeStruct((N, O, HW), jnp.float32),
        scratch_shapes=[pltpu.VMEM((O, 1), jnp.float32)],
    )(x, w2, b2)
    return out.reshape(N, O, H, Wd)


# speedup vs baseline: 1.4415x; 1.4415x over previous
import jax
import jax.numpy as jnp
from jax.experimental import pallas as pl

_S = 6


def _pool_conv_broadcast_kernel(*refs):
    xs = refs[:_S]
    w_ref, b_ref, o_ref = refs[_S], refs[_S + 1], refs[_S + 2]
    ms = [jnp.sum(x[0], axis=1, keepdims=True) for x in xs]
    m = jnp.concatenate(ms, axis=0)
    hw = o_ref.shape[2]
    pooled = jnp.dot(w_ref[...], m * (1.0 / hw),
                     preferred_element_type=jnp.float32) + b_ref[...]
    o_ref[0] = jnp.broadcast_to(pooled, o_ref.shape[1:])


def kernel(features, depth, W, b):
    del depth
    N, C, H, Wd = features.shape
    O = W.shape[0]
    HW = H * Wd
    CK = C // _S
    x = features.reshape(N, C, HW)
    w2 = W.reshape(O, C)
    b2 = b.reshape(O, 1)
    x_specs = [
        pl.BlockSpec((1, CK, HW), lambda i, s=s: (i, s, 0)) for s in range(_S)
    ]
    out = pl.pallas_call(
        _pool_conv_broadcast_kernel,
        grid=(N,),
        in_specs=x_specs + [
            pl.BlockSpec((O, C), lambda i: (0, 0)),
            pl.BlockSpec((O, 1), lambda i: (0, 0)),
        ],
        out_specs=pl.BlockSpec((1, O, HW), lambda i: (i, 0, 0)),
        out_shape=jax.ShapeDtypeStruct((N, O, HW), jnp.float32),
    )(*([x] * _S), w2, b2)
    return out.reshape(N, O, H, Wd)
